# baseline (speedup 1.0000x reference)
import jax
import jax.numpy as jnp
from jax.experimental import pallas as pl
from jax.experimental.pallas import tpu as pltpu

_B, _N = 8, 2048
_BR = 256
_NBLK = _N // _BR
_EPS = 1e-12
_HI = jax.lax.Precision.HIGHEST


def _bf(x):
    return x.astype(jnp.bfloat16)


def _sel_dist(rows_t, cols):
    bb = jnp.sum(rows_t * rows_t, axis=0, keepdims=True)
    aa = jnp.sum(cols * cols, axis=1, keepdims=True)
    ab = jax.lax.dot_general(
        _bf(cols), _bf(rows_t), (((1,), (0,)), ((), ())),
        preferred_element_type=jnp.float32)
    return (aa + bb) - 2.0 * ab


def _argmin_mask(dmat, iota):
    dmin = jnp.min(dmat, axis=1, keepdims=True)
    jmin = jnp.min(jnp.where(dmat == dmin, iota, _N), axis=1, keepdims=True)
    return iota == jmin


def _onehot_pick(mask, table):
    return jax.lax.dot_general(
        mask.astype(jnp.float32), table, (((1,), (0,)), ((), ())),
        precision=_HI, preferred_element_type=jnp.float32)


def _contrib(mask, full, cols, nrm):
    p = _onehot_pick(mask, full)
    dx = p[:, 0:1] - cols[:, 0:1]
    dy = p[:, 1:2] - cols[:, 1:2]
    dz = p[:, 2:3] - cols[:, 2:3]
    norm = jnp.sqrt(dx * dx + dy * dy + dz * dz)
    inv = 1.0 / jnp.maximum(norm, _EPS)
    return jnp.abs((dx * inv) * nrm[:, 0:1] + (dy * inv) * nrm[:, 1:2]
                   + (dz * inv) * nrm[:, 2:3])


def _cloud_kappa(rows_t, full, cols, nrm, iota):
    inf = jnp.float32(jnp.inf)
    dsel = _sel_dist(rows_t, cols)
    m1 = _argmin_mask(dsel, iota)
    dm = jnp.where(m1, inf, dsel)
    m2 = _argmin_mask(dm, iota)
    dm2 = jnp.where(m2, inf, dm)
    m3 = _argmin_mask(dm2, iota)
    return (_contrib(m2, full, cols, nrm) + _contrib(m3, full, cols, nrm)) * 0.5


def _body(ot_ref, at_ref, of_ref, af_ref, nf_ref, o_ref, a_ref, n_ref,
          out_ref, acc_ref):
    r = pl.program_id(1)
    pt = ot_ref[0]
    at = at_ref[0]
    pfull = of_ref[0]
    afull = af_ref[0]
    nfull = nf_ref[0]
    ob = o_ref[0]
    ab_ = a_ref[0]
    nb = n_ref[0]
    iota = jax.lax.broadcasted_iota(jnp.int32, (_BR, _N), 1)

    @pl.when(r == 0)
    def _():
        acc_ref[0] = 0.0
        acc_ref[1] = 0.0
        acc_ref[2] = 0.0
        acc_ref[3] = 0.0

    ko = _cloud_kappa(pt, pfull, ob, nb, iota)

    dao = _sel_dist(pt, ab_)
    nh = _onehot_pick(_argmin_mask(dao, iota), nfull)

    ka = _cloud_kappa(at, afull, ab_, nh, iota)

    acc_ref[0] += jnp.sum(ko)
    acc_ref[1] += jnp.sum(ko * ko)
    acc_ref[2] += jnp.sum(ka)
    acc_ref[3] += jnp.sum(ka * ka)

    @pl.when(r == _NBLK - 1)
    def _():
        n = jnp.float32(_N)
        var_o = (acc_ref[1] - acc_ref[0] * acc_ref[0] / n) / (n - 1.0)
        var_a = (acc_ref[3] - acc_ref[2] * acc_ref[2] / n) / (n - 1.0)
        std_o = jnp.sqrt(jnp.maximum(var_o, 0.0))
        std_a = jnp.sqrt(jnp.maximum(var_a, 0.0))
        out_ref[...] = jnp.full((1, 1, 1), jnp.abs(std_a - std_o), jnp.float32)


def _mean_body(x_ref, o_ref):
    o_ref[...] = jnp.sum(x_ref[...], axis=0, keepdims=True) / jnp.float32(_B)


def _call(ori_pcs, adv_pcs, ori_normals, interpret=False):
    ori_t = ori_pcs.transpose(0, 2, 1)
    adv_t = adv_pcs.transpose(0, 2, 1)
    per_batch = pl.pallas_call(
        _body,
        grid=(_B, _NBLK),
        in_specs=[
            pl.BlockSpec((1, 3, _N), lambda b, r: (b, 0, 0)),
            pl.BlockSpec((1, 3, _N), lambda b, r: (b, 0, 0)),
            pl.BlockSpec((1, _N, 3), lambda b, r: (b, 0, 0)),
            pl.BlockSpec((1, _N, 3), lambda b, r: (b, 0, 0)),
            pl.BlockSpec((1, _N, 3), lambda b, r: (b, 0, 0)),
            pl.BlockSpec((1, _BR, 3), lambda b, r: (b, r, 0)),
            pl.BlockSpec((1, _BR, 3), lambda b, r: (b, r, 0)),
            pl.BlockSpec((1, _BR, 3), lambda b, r: (b, r, 0)),
        ],
        out_specs=pl.BlockSpec((1, 1, 1), lambda b, r: (b, 0, 0)),
        out_shape=jax.ShapeDtypeStruct((_B, 1, 1), jnp.float32),
        scratch_shapes=[pltpu.SMEM((4,), jnp.float32)],
        compiler_params=pltpu.CompilerParams(
            dimension_semantics=("parallel", "arbitrary")),
        interpret=interpret,
    )(ori_t, adv_t, ori_pcs, adv_pcs, ori_normals,
      ori_pcs, adv_pcs, ori_normals)
    out = pl.pallas_call(
        _mean_body,
        out_shape=jax.ShapeDtypeStruct((1, 1, 1), jnp.float32),
        interpret=interpret,
    )(per_batch)
    return out[0, 0, 0]


def kernel(ori_pcs, adv_pcs, ori_normals):
    return _call(ori_pcs, adv_pcs, ori_normals)


# speedup vs baseline: 1.9124x; 1.9124x over previous
import jax
import jax.numpy as jnp
from jax.experimental import pallas as pl
from jax.experimental.pallas import tpu as pltpu

_B, _N = 8, 2048
_BR = 256
_NBLK = _N // _BR
_EPS = 1e-12
_HI = jax.lax.Precision.HIGHEST


def _bf(x):
    return x.astype(jnp.bfloat16)


def _sel_dist(rows_t, cols):
    bb = jnp.sum(rows_t * rows_t, axis=0, keepdims=True)
    aa = jnp.sum(cols * cols, axis=1, keepdims=True)
    ab = jax.lax.dot_general(
        _bf(cols), _bf(rows_t), (((1,), (0,)), ((), ())),
        preferred_element_type=jnp.float32)
    return (aa + bb) - 2.0 * ab


def _argmin_mask(dmat, iota):
    jmin = jnp.argmin(dmat, axis=1).astype(jnp.int32)[:, None]
    return iota == jmin


def _extract(mask, rows):
    return [jnp.sum(jnp.where(mask, rows[d], 0.0), axis=1, keepdims=True)
            for d in range(3)]


def _contrib(mask, rows_t, cols, nrm):
    px, py, pz = _extract(mask, [rows_t[d:d + 1, :] for d in range(3)])
    dx = px - cols[:, 0:1]
    dy = py - cols[:, 1:2]
    dz = pz - cols[:, 2:3]
    norm = jnp.sqrt(dx * dx + dy * dy + dz * dz)
    inv = 1.0 / jnp.maximum(norm, _EPS)
    return jnp.abs((dx * inv) * nrm[:, 0:1] + (dy * inv) * nrm[:, 1:2]
                   + (dz * inv) * nrm[:, 2:3])


def _cloud_kappa(rows_t, cols, nrm, iota):
    inf = jnp.float32(jnp.inf)
    dsel = _sel_dist(rows_t, cols)
    m1 = _argmin_mask(dsel, iota)
    dm = jnp.where(m1, inf, dsel)
    m2 = _argmin_mask(dm, iota)
    dm2 = jnp.where(m2, inf, dm)
    m3 = _argmin_mask(dm2, iota)
    return (_contrib(m2, rows_t, cols, nrm) + _contrib(m3, rows_t, cols, nrm)) * 0.5


def _body(ot_ref, at_ref, nt_ref, o_ref, a_ref, n_ref, out_ref, acc_ref):
    r = pl.program_id(1)
    pt = ot_ref[0]
    at = at_ref[0]
    nt = nt_ref[0]
    ob = o_ref[0]
    ab_ = a_ref[0]
    nb = n_ref[0]
    iota = jax.lax.broadcasted_iota(jnp.int32, (_BR, _N), 1)

    @pl.when(r == 0)
    def _():
        acc_ref[0] = 0.0
        acc_ref[1] = 0.0
        acc_ref[2] = 0.0
        acc_ref[3] = 0.0

    ko = _cloud_kappa(pt, ob, nb, iota)

    dao = _sel_dist(pt, ab_)
    nhx, nhy, nhz = _extract(_argmin_mask(dao, iota),
                             [nt[d:d + 1, :] for d in range(3)])
    nh = jnp.concatenate([nhx, nhy, nhz], axis=1)

    ka = _cloud_kappa(at, ab_, nh, iota)

    acc_ref[0] += jnp.sum(ko)
    acc_ref[1] += jnp.sum(ko * ko)
    acc_ref[2] += jnp.sum(ka)
    acc_ref[3] += jnp.sum(ka * ka)

    @pl.when(r == _NBLK - 1)
    def _():
        n = jnp.float32(_N)
        var_o = (acc_ref[1] - acc_ref[0] * acc_ref[0] / n) / (n - 1.0)
        var_a = (acc_ref[3] - acc_ref[2] * acc_ref[2] / n) / (n - 1.0)
        std_o = jnp.sqrt(jnp.maximum(var_o, 0.0))
        std_a = jnp.sqrt(jnp.maximum(var_a, 0.0))
        out_ref[...] = jnp.full((1, 1, 1), jnp.abs(std_a - std_o), jnp.float32)


def _mean_body(x_ref, o_ref):
    o_ref[...] = jnp.sum(x_ref[...], axis=0, keepdims=True) / jnp.float32(_B)


def _call(ori_pcs, adv_pcs, ori_normals, interpret=False):
    ori_t = ori_pcs.transpose(0, 2, 1)
    adv_t = adv_pcs.transpose(0, 2, 1)
    nrm_t = ori_normals.transpose(0, 2, 1)
    per_batch = pl.pallas_call(
        _body,
        grid=(_B, _NBLK),
        in_specs=[
            pl.BlockSpec((1, 3, _N), lambda b, r: (b, 0, 0)),
            pl.BlockSpec((1, 3, _N), lambda b, r: (b, 0, 0)),
            pl.BlockSpec((1, 3, _N), lambda b, r: (b, 0, 0)),
            pl.BlockSpec((1, _BR, 3), lambda b, r: (b, r, 0)),
            pl.BlockSpec((1, _BR, 3), lambda b, r: (b, r, 0)),
            pl.BlockSpec((1, _BR, 3), lambda b, r: (b, r, 0)),
        ],
        out_specs=pl.BlockSpec((1, 1, 1), lambda b, r: (b, 0, 0)),
        out_shape=jax.ShapeDtypeStruct((_B, 1, 1), jnp.float32),
        scratch_shapes=[pltpu.SMEM((4,), jnp.float32)],
        compiler_params=pltpu.CompilerParams(
            dimension_semantics=("parallel", "arbitrary")),
        interpret=interpret,
    )(ori_t, adv_t, nrm_t, ori_pcs, adv_pcs, ori_normals)
    out = pl.pallas_call(
        _mean_body,
        out_shape=jax.ShapeDtypeStruct((1, 1, 1), jnp.float32),
        interpret=interpret,
    )(per_batch)
    return out[0, 0, 0]


def kernel(ori_pcs, adv_pcs, ori_normals):
    return _call(ori_pcs, adv_pcs, ori_normals)


# speedup vs baseline: 2.6947x; 1.4091x over previous
import jax
import jax.numpy as jnp
from jax.experimental import pallas as pl
from jax.experimental.pallas import tpu as pltpu

_B, _N = 8, 2048
_BR = 512
_NBLK = _N // _BR
_EPS = 1e-12


def _bf(x):
    return x.astype(jnp.bfloat16)


def _sel_dist(rows_t, cols):
    bb = jnp.sum(rows_t * rows_t, axis=0, keepdims=True)
    aa = jnp.sum(cols * cols, axis=1, keepdims=True)
    ab = jax.lax.dot_general(
        _bf(cols), _bf(rows_t), (((1,), (0,)), ((), ())),
        preferred_element_type=jnp.float32)
    return (aa + bb) - 2.0 * ab


def _argmin_mask(dmat, iota):
    jmin = jnp.argmin(dmat, axis=1).astype(jnp.int32)[:, None]
    return iota == jmin


def _pick(mask, tbl):
    e = jax.lax.dot_general(
        _bf(mask), tbl, (((1,), (0,)), ((), ())),
        preferred_element_type=jnp.float32)
    return ((e[:, 0:1] + e[:, 1:2]) + e[:, 2:3],
            (e[:, 3:4] + e[:, 4:5]) + e[:, 5:6],
            (e[:, 6:7] + e[:, 7:8]) + e[:, 8:9])


def _contrib(mask, tbl, cols, nrm):
    px, py, pz = _pick(mask, tbl)
    dx = px - cols[:, 0:1]
    dy = py - cols[:, 1:2]
    dz = pz - cols[:, 2:3]
    norm = jnp.sqrt(dx * dx + dy * dy + dz * dz)
    inv = 1.0 / jnp.maximum(norm, _EPS)
    return jnp.abs((dx * inv) * nrm[:, 0:1] + (dy * inv) * nrm[:, 1:2]
                   + (dz * inv) * nrm[:, 2:3])


def _cloud_kappa(rows_t, tbl, cols, nrm, iota):
    inf = jnp.float32(jnp.inf)
    dsel = _sel_dist(rows_t, cols)
    m1 = _argmin_mask(dsel, iota)
    dm = jnp.where(m1, inf, dsel)
    m2 = _argmin_mask(dm, iota)
    dm2 = jnp.where(m2, inf, dm)
    m3 = _argmin_mask(dm2, iota)
    return (_contrib(m2, tbl, cols, nrm) + _contrib(m3, tbl, cols, nrm)) * 0.5


def _body(ot_ref, at_ref, to_ref, ta_ref, tn_ref, o_ref, a_ref, n_ref,
          out_ref, acc_ref):
    r = pl.program_id(1)
    pt = ot_ref[0]
    at = at_ref[0]
    tblo = to_ref[0]
    tbla = ta_ref[0]
    tbln = tn_ref[0]
    ob = o_ref[0]
    ab_ = a_ref[0]
    nb = n_ref[0]
    iota = jax.lax.broadcasted_iota(jnp.int32, (_BR, _N), 1)

    @pl.when(r == 0)
    def _():
        acc_ref[0] = 0.0
        acc_ref[1] = 0.0
        acc_ref[2] = 0.0
        acc_ref[3] = 0.0

    ko = _cloud_kappa(pt, tblo, ob, nb, iota)

    dao = _sel_dist(pt, ab_)
    nhx, nhy, nhz = _pick(_argmin_mask(dao, iota), tbln)
    nh = jnp.concatenate([nhx, nhy, nhz], axis=1)

    ka = _cloud_kappa(at, tbla, ab_, nh, iota)

    acc_ref[0] += jnp.sum(ko)
    acc_ref[1] += jnp.sum(ko * ko)
    acc_ref[2] += jnp.sum(ka)
    acc_ref[3] += jnp.sum(ka * ka)

    @pl.when(r == _NBLK - 1)
    def _():
        n = jnp.float32(_N)
        var_o = (acc_ref[1] - acc_ref[0] * acc_ref[0] / n) / (n - 1.0)
        var_a = (acc_ref[3] - acc_ref[2] * acc_ref[2] / n) / (n - 1.0)
        std_o = jnp.sqrt(jnp.maximum(var_o, 0.0))
        std_a = jnp.sqrt(jnp.maximum(var_a, 0.0))
        out_ref[...] = jnp.full((1, 1, 1), jnp.abs(std_a - std_o), jnp.float32)


def _mean_body(x_ref, o_ref):
    o_ref[...] = jnp.sum(x_ref[...], axis=0, keepdims=True) / jnp.float32(_B)


def _limbs(x):
    hi = x.astype(jnp.bfloat16)
    r1 = x - hi.astype(jnp.float32)
    mid = r1.astype(jnp.bfloat16)
    lo = (r1 - mid.astype(jnp.float32)).astype(jnp.bfloat16)
    return jnp.stack([hi, mid, lo], axis=-1).reshape(x.shape[0], x.shape[1], 9)


def _call(ori_pcs, adv_pcs, ori_normals, interpret=False):
    ori_t = ori_pcs.transpose(0, 2, 1)
    adv_t = adv_pcs.transpose(0, 2, 1)
    per_batch = pl.pallas_call(
        _body,
        grid=(_B, _NBLK),
        in_specs=[
            pl.BlockSpec((1, 3, _N), lambda b, r: (b, 0, 0)),
            pl.BlockSpec((1, 3, _N), lambda b, r: (b, 0, 0)),
            pl.BlockSpec((1, _N, 9), lambda b, r: (b, 0, 0)),
            pl.BlockSpec((1, _N, 9), lambda b, r: (b, 0, 0)),
            pl.BlockSpec((1, _N, 9), lambda b, r: (b, 0, 0)),
            pl.BlockSpec((1, _BR, 3), lambda b, r: (b, r, 0)),
            pl.BlockSpec((1, _BR, 3), lambda b, r: (b, r, 0)),
            pl.BlockSpec((1, _BR, 3), lambda b, r: (b, r, 0)),
        ],
        out_specs=pl.BlockSpec((1, 1, 1), lambda b, r: (b, 0, 0)),
        out_shape=jax.ShapeDtypeStruct((_B, 1, 1), jnp.float32),
        scratch_shapes=[pltpu.SMEM((4,), jnp.float32)],
        compiler_params=pltpu.CompilerParams(
            dimension_semantics=("parallel", "arbitrary")),
        interpret=interpret,
    )(ori_t, adv_t, _limbs(ori_pcs), _limbs(adv_pcs), _limbs(ori_normals),
      ori_pcs, adv_pcs, ori_normals)
    out = pl.pallas_call(
        _mean_body,
        out_shape=jax.ShapeDtypeStruct((1, 1, 1), jnp.float32),
        interpret=interpret,
    )(per_batch)
    return out[0, 0, 0]


def kernel(ori_pcs, adv_pcs, ori_normals):
    return _call(ori_pcs, adv_pcs, ori_normals)
